# Initial kernel scaffold; baseline (speedup 1.0000x reference)
import jax
import jax.numpy as jnp
from jax.experimental import pallas as pl


def kernel(indices, emb_weight):
    raise NotImplementedError("write your pallas kernel here")


# speedup vs baseline: 2.6912x; 2.6912x over previous
import functools

import jax
import jax.numpy as jnp
from jax import lax
from jax.experimental import pallas as pl
from jax.experimental.pallas import tpu as pltpu
from jax.experimental.pallas import tpu_sc as plsc

_B = 16384
_PAD = 50
_D = 64
_TOTAL = _B * _PAD
_NC, _NS = 2, 16
_NW = _NC * _NS
_PER_W = _TOTAL // _NW
_C = 128
_NCHUNK = _PER_W // _C


def _make_emb():
    mesh = plsc.VectorSubcoreMesh(core_axis_name="c", subcore_axis_name="s")

    @functools.partial(
        pl.kernel,
        mesh=mesh,
        out_type=jax.ShapeDtypeStruct((_TOTAL, _D), jnp.float32),
        scratch_types=[
            pltpu.VMEM((1, _C), jnp.int32),
            pltpu.VMEM((_C, _D), jnp.float32),
            pltpu.SemaphoreType.DMA,
        ],
        compiler_params=pltpu.CompilerParams(use_tc_tiling_on_sc=False),
    )
    def emb(idx_hbm, table_hbm, out_hbm, idx_v, rows_v, sem):
        wid = lax.axis_index("s") * _NC + lax.axis_index("c")
        base = wid * _PER_W

        def body(g, carry):
            off = base + g * _C
            pltpu.sync_copy(idx_hbm.at[pl.ds(off, _C)], idx_v.at[0])
            pltpu.async_copy(table_hbm.at[idx_v.at[0]], rows_v, sem).wait()
            pltpu.sync_copy(rows_v, out_hbm.at[pl.ds(off, _C)])
            return carry

        lax.fori_loop(0, _NCHUNK, body, 0)

    return emb


_emb = _make_emb()


@jax.jit
def kernel(indices, emb_weight):
    flat = indices.reshape(_TOTAL)
    out = _emb(flat, emb_weight)
    return out.reshape(_B, _PAD, _D)
